# initial kernel scaffold (unmeasured)
import jax
import jax.numpy as jnp
from jax import lax
from jax.experimental import pallas as pl
from jax.experimental.pallas import tpu as pltpu


def kernel(
    x,
):
    def body(*refs):
        pass

    out_shape = jax.ShapeDtypeStruct(..., jnp.float32)
    return pl.pallas_call(body, out_shape=out_shape)(...)


# baseline (device time: 426428 ns/iter reference)
import jax
import jax.numpy as jnp
from jax import lax
from jax.experimental import pallas as pl
from jax.experimental.pallas import tpu as pltpu

M = 8192
N = 1024
CH = 2048
N_CHUNKS = M // CH


def kernel(x):
    x = x.reshape(M, 2 * N)

    def body(x_hbm, out_hbm, recv_hbm, own_v, recv_v, out_v,
             copy_sems, send_sem, recv_sem):
        my_x = lax.axis_index("x")
        my_y = lax.axis_index("y")
        peer_y = 1 - my_y

        barrier_sem = pltpu.get_barrier_semaphore()
        pl.semaphore_signal(
            barrier_sem, inc=1,
            device_id=(my_x, peer_y), device_id_type=pl.DeviceIdType.MESH,
        )
        pl.semaphore_wait(barrier_sem, 1)

        rdma = pltpu.make_async_remote_copy(
            src_ref=x_hbm.at[:, pl.ds(peer_y * N, N)],
            dst_ref=recv_hbm,
            send_sem=send_sem,
            recv_sem=recv_sem,
            device_id=(my_x, peer_y),
            device_id_type=pl.DeviceIdType.MESH,
        )
        rdma.start()
        rdma.wait()

        for k in range(N_CHUNKS):
            rows = pl.ds(k * CH, CH)
            c_own = pltpu.make_async_copy(
                x_hbm.at[rows, pl.ds(my_y * N, N)], own_v, copy_sems.at[0]
            )
            c_recv = pltpu.make_async_copy(
                recv_hbm.at[rows], recv_v, copy_sems.at[1]
            )
            c_own.start()
            c_recv.start()
            c_own.wait()
            c_recv.wait()
            out_v[:, :] = own_v[:, :] + recv_v[:, :]
            c_out = pltpu.make_async_copy(out_v, out_hbm.at[rows], copy_sems.at[2])
            c_out.start()
            c_out.wait()

    out, _ = pl.pallas_call(
        body,
        out_shape=[
            jax.ShapeDtypeStruct((M, N), jnp.float32),
            jax.ShapeDtypeStruct((M, N), jnp.float32),
        ],
        in_specs=[pl.BlockSpec(memory_space=pltpu.MemorySpace.HBM)],
        out_specs=[
            pl.BlockSpec(memory_space=pltpu.MemorySpace.HBM),
            pl.BlockSpec(memory_space=pltpu.MemorySpace.HBM),
        ],
        scratch_shapes=[
            pltpu.VMEM((CH, N), jnp.float32),
            pltpu.VMEM((CH, N), jnp.float32),
            pltpu.VMEM((CH, N), jnp.float32),
            pltpu.SemaphoreType.DMA((3,)),
            pltpu.SemaphoreType.DMA,
            pltpu.SemaphoreType.DMA,
        ],
        compiler_params=pltpu.CompilerParams(collective_id=0),
    )(x)
    return out


# device time: 216812 ns/iter; 1.9668x vs baseline; 1.9668x over previous
import jax
import jax.numpy as jnp
from jax import lax
from jax.experimental import pallas as pl
from jax.experimental.pallas import tpu as pltpu

M = 8192
N = 1024
H = M // 2
K = 32
CHR = H // K


def kernel(x):
    x = x.reshape(M, 2 * N)

    def body(x_hbm, out_hbm, own_v, recv_v, res_v,
             y_send_sems, y_recv_sems, x_send_sems, x_recv_sems,
             own_sem, store_sem):
        my_x = lax.axis_index("x")
        my_y = lax.axis_index("y")
        peer_y = 1 - my_y
        peer_x = 1 - my_x
        row_base = my_x * H

        barrier_sem = pltpu.get_barrier_semaphore()
        for dev in [(my_x, peer_y), (peer_x, my_y)]:
            pl.semaphore_signal(
                barrier_sem, inc=1,
                device_id=dev, device_id_type=pl.DeviceIdType.MESH,
            )
        pl.semaphore_wait(barrier_sem, 2)

        own_copy = pltpu.make_async_copy(
            x_hbm.at[pl.ds(row_base, H), pl.ds(my_y * N, N)], own_v, own_sem
        )
        own_copy.start()

        y_rdmas = []
        for k in range(K):
            r = pltpu.make_async_remote_copy(
                src_ref=x_hbm.at[pl.ds(row_base + k * CHR, CHR),
                                 pl.ds(peer_y * N, N)],
                dst_ref=recv_v.at[pl.ds(k * CHR, CHR), :],
                send_sem=y_send_sems.at[k],
                recv_sem=y_recv_sems.at[k],
                device_id=(my_x, peer_y),
                device_id_type=pl.DeviceIdType.MESH,
            )
            r.start()
            y_rdmas.append(r)

        own_copy.wait()

        x_rdmas = []
        for k in range(K):
            rows = pl.ds(k * CHR, CHR)
            y_rdmas[k].wait_recv()
            res_v[rows, :] = own_v[rows, :] + recv_v[rows, :]
            r = pltpu.make_async_remote_copy(
                src_ref=res_v.at[rows, :],
                dst_ref=out_hbm.at[pl.ds(row_base + k * CHR, CHR), :],
                send_sem=x_send_sems.at[k],
                recv_sem=x_recv_sems.at[k],
                device_id=(peer_x, my_y),
                device_id_type=pl.DeviceIdType.MESH,
            )
            r.start()
            x_rdmas.append(r)

        store = pltpu.make_async_copy(
            res_v, out_hbm.at[pl.ds(row_base, H), :], store_sem
        )
        store.start()

        for k in range(K):
            y_rdmas[k].wait_send()
            x_rdmas[k].wait_send()
            x_rdmas[k].wait_recv()
        store.wait()

    out = pl.pallas_call(
        body,
        out_shape=jax.ShapeDtypeStruct((M, N), jnp.float32),
        in_specs=[pl.BlockSpec(memory_space=pltpu.MemorySpace.HBM)],
        out_specs=pl.BlockSpec(memory_space=pltpu.MemorySpace.HBM),
        scratch_shapes=[
            pltpu.VMEM((H, N), jnp.float32),
            pltpu.VMEM((H, N), jnp.float32),
            pltpu.VMEM((H, N), jnp.float32),
            pltpu.SemaphoreType.DMA((K,)),
            pltpu.SemaphoreType.DMA((K,)),
            pltpu.SemaphoreType.DMA((K,)),
            pltpu.SemaphoreType.DMA((K,)),
            pltpu.SemaphoreType.DMA,
            pltpu.SemaphoreType.DMA,
        ],
        compiler_params=pltpu.CompilerParams(
            collective_id=0,
            vmem_limit_bytes=60 * 1024 * 1024,
        ),
    )(x)
    return out
